# Initial kernel scaffold; baseline (speedup 1.0000x reference)
import jax
import jax.numpy as jnp
from jax.experimental import pallas as pl


def kernel(x, edge_index, W1, b1, W2, b2):
    raise NotImplementedError("write your pallas kernel here")


# speedup vs baseline: 19.8151x; 19.8151x over previous
import functools

import jax
import jax.numpy as jnp
from jax import lax
from jax.experimental import pallas as pl
from jax.experimental.pallas import tpu as pltpu
from jax.experimental.pallas import tpu_sc as plsc

_NC = 2
_NS = 16
_NW = _NC * _NS
_CHUNK = 80


def _make_propagate(n, d, n_chunks):
    mesh = plsc.VectorSubcoreMesh(core_axis_name="c", subcore_axis_name="s")
    rows_main = (n // _NS) // 8 * 8
    rem = n - rows_main * _NS

    @functools.partial(
        pl.kernel,
        out_type=jax.ShapeDtypeStruct((_NC, n, d), jnp.float32),
        mesh=mesh,
        compiler_params=pltpu.CompilerParams(use_tc_tiling_on_sc=False),
        scratch_types=[
            pltpu.VMEM((n_chunks, _CHUNK), jnp.int32),
            pltpu.VMEM((n_chunks, _CHUNK), jnp.int32),
            pltpu.VMEM((_CHUNK, d), jnp.float32),
            pltpu.VMEM_SHARED((n, d), jnp.float32),
        ],
    )
    def prop(xs_hbm, src_hbm, dst_hbm, out_hbm, sidx, didx, rows, acc):
        c = lax.axis_index("c")
        s = lax.axis_index("s")
        wid = c * _NS + s
        base = s * rows_main

        pltpu.sync_copy(xs_hbm.at[pl.ds(base, rows_main)],
                        acc.at[pl.ds(base, rows_main)])
        if rem:
            @pl.when(s == _NS - 1)
            def _():
                pltpu.sync_copy(xs_hbm.at[pl.ds(_NS * rows_main, rem)],
                                acc.at[pl.ds(_NS * rows_main, rem)])

        pltpu.sync_copy(src_hbm.at[wid], sidx)
        pltpu.sync_copy(dst_hbm.at[wid], didx)
        plsc.subcore_barrier()

        def body(k, carry):
            pltpu.sync_copy(xs_hbm.at[sidx.at[k]], rows)
            pltpu.sync_copy(rows, acc.at[didx.at[k]], add=True)
            return carry

        lax.fori_loop(0, n_chunks, body, 0)
        plsc.subcore_barrier()

        pltpu.sync_copy(acc.at[pl.ds(base, rows_main)],
                        out_hbm.at[c, pl.ds(base, rows_main)])
        if rem:
            @pl.when(s == _NS - 1)
            def _():
                pltpu.sync_copy(acc.at[pl.ds(_NS * rows_main, rem)],
                                out_hbm.at[c, pl.ds(_NS * rows_main, rem)])

    return prop


def _tc_prep(x, w1, degs, blk=1000):
    n, din = x.shape
    h = w1.shape[1]
    dw = degs.shape[2]

    def body(x_ref, w1_ref, degs_ref, xs_ref, dis_ref):
        deg = degs_ref[0, :, 0:1] + degs_ref[1, :, 0:1] - 1.0
        dis = lax.rsqrt(deg)
        xw = lax.dot_general(x_ref[...], w1_ref[...], (((1,), (0,)), ((), ())),
                             preferred_element_type=jnp.float32)
        xs_ref[...] = xw * dis
        dis_ref[...] = dis

    return pl.pallas_call(
        body,
        grid=(n // blk,),
        in_specs=[
            pl.BlockSpec((blk, din), lambda i: (i, 0)),
            pl.BlockSpec((din, h), lambda i: (0, 0)),
            pl.BlockSpec((_NC, blk, dw), lambda i: (0, i, 0)),
        ],
        out_specs=[
            pl.BlockSpec((blk, h), lambda i: (i, 0)),
            pl.BlockSpec((blk, 1), lambda i: (i, 0)),
        ],
        out_shape=[
            jax.ShapeDtypeStruct((n, h), jnp.float32),
            jax.ShapeDtypeStruct((n, 1), jnp.float32),
        ],
    )(x, w1, degs)


def _tc_mid(accs, xs, dis, b1r, w2p, blk=1000):
    n, h = xs.shape
    d2 = w2p.shape[1]

    def body(a_ref, xs_ref, dis_ref, b1_ref, w2_ref, hs_ref):
        ssum = a_ref[0] + a_ref[1] - xs_ref[...]
        hval = jnp.maximum(ssum * dis_ref[...] + b1_ref[...], 0.0)
        h2 = lax.dot_general(hval, w2_ref[...], (((1,), (0,)), ((), ())),
                             preferred_element_type=jnp.float32)
        hs_ref[...] = h2 * dis_ref[...]

    return pl.pallas_call(
        body,
        grid=(n // blk,),
        in_specs=[
            pl.BlockSpec((_NC, blk, h), lambda i: (0, i, 0)),
            pl.BlockSpec((blk, h), lambda i: (i, 0)),
            pl.BlockSpec((blk, 1), lambda i: (i, 0)),
            pl.BlockSpec((1, h), lambda i: (0, 0)),
            pl.BlockSpec((h, d2), lambda i: (0, 0)),
        ],
        out_specs=pl.BlockSpec((blk, d2), lambda i: (i, 0)),
        out_shape=jax.ShapeDtypeStruct((n, d2), jnp.float32),
    )(accs, xs, dis, b1r, w2p)


def _tc_final(accs2, hs, dis, b2p, blk=1000):
    n, d2 = hs.shape

    def body(a_ref, hs_ref, dis_ref, b2_ref, out_ref):
        ssum = a_ref[0] + a_ref[1] - hs_ref[...]
        out_ref[...] = ssum * dis_ref[...] + b2_ref[...]

    return pl.pallas_call(
        body,
        grid=(n // blk,),
        in_specs=[
            pl.BlockSpec((_NC, blk, d2), lambda i: (0, i, 0)),
            pl.BlockSpec((blk, d2), lambda i: (i, 0)),
            pl.BlockSpec((blk, 1), lambda i: (i, 0)),
            pl.BlockSpec((1, d2), lambda i: (0, 0)),
        ],
        out_specs=pl.BlockSpec((blk, d2), lambda i: (i, 0)),
        out_shape=jax.ShapeDtypeStruct((n, d2), jnp.float32),
    )(accs2, hs, dis, b2p)


def kernel(x, edge_index, W1, b1, W2, b2):
    n, din = x.shape
    h = W1.shape[1]
    out_w = W2.shape[1]
    e = edge_index.shape[1]

    src = edge_index[0].astype(jnp.int32)
    dst = edge_index[1].astype(jnp.int32)
    n_chunks = e // (_NW * _CHUNK)
    src3 = src.reshape(_NW, n_chunks, _CHUNK)
    dst3 = dst.reshape(_NW, n_chunks, _CHUNK)

    dpad = 16
    ones16 = jnp.ones((n, dpad), jnp.float32)
    w2p = jnp.pad(W2, ((0, 0), (0, dpad - out_w)))
    b2p = jnp.pad(b2, (0, dpad - out_w)).reshape(1, dpad)
    b1r = b1.reshape(1, h)

    prop16 = _make_propagate(n, dpad, n_chunks)
    prop_h = _make_propagate(n, h, n_chunks)

    degs = prop16(ones16, src3, dst3)
    xs, dis = _tc_prep(x, W1, degs)
    accs = prop_h(xs, src3, dst3)
    hs = _tc_mid(accs, xs, dis, b1r, w2p)
    accs2 = prop16(hs, src3, dst3)
    out16 = _tc_final(accs2, hs, dis, b2p)
    return out16[:, :out_w]
